# Initial kernel scaffold; baseline (speedup 1.0000x reference)
import jax
import jax.numpy as jnp
from jax.experimental import pallas as pl


def kernel(x, levels):
    raise NotImplementedError("write your pallas kernel here")


# speedup vs baseline: 1.0610x; 1.0610x over previous
import jax
import jax.numpy as jnp
from jax.experimental import pallas as pl
from jax.experimental.pallas import tpu as pltpu


def _quantize_block(lv_ref, x_ref, o_ref):
    l0, l1, l2 = lv_ref[0], lv_ref[1], lv_ref[2]
    t0 = (l0 + l1) * 0.5
    t1 = (l1 + l2) * 0.5
    x = x_ref[...]
    o_ref[...] = jnp.where(x > t1, l2, jnp.where(x > t0, l1, l0))


def kernel(x, levels):
    b, c, d = x.shape
    rows = b * c
    x2 = x.reshape(rows, d)
    block_rows = 256
    out = pl.pallas_call(
        _quantize_block,
        grid=(rows // block_rows,),
        in_specs=[
            pl.BlockSpec(memory_space=pltpu.MemorySpace.SMEM),
            pl.BlockSpec((block_rows, d), lambda i: (i, 0)),
        ],
        out_specs=pl.BlockSpec((block_rows, d), lambda i: (i, 0)),
        out_shape=jax.ShapeDtypeStruct((rows, d), x.dtype),
    )(levels, x2)
    return out.reshape(b, c, d)
